# Initial kernel scaffold; baseline (speedup 1.0000x reference)
import jax
import jax.numpy as jnp
from jax.experimental import pallas as pl


def kernel(images, theta):
    raise NotImplementedError("write your pallas kernel here")


# speedup vs baseline: 1.3752x; 1.3752x over previous
import functools

import jax
import jax.numpy as jnp
from jax import lax
from jax.experimental import pallas as pl
from jax.experimental.pallas import tpu as pltpu
from jax.experimental.pallas import tpu_sc as plsc

B, H, W, C = 8, 224, 224, 96
N = B * H * W
HW = H * W
NW = 32
ROWS_PER_TILE = (B * H) // NW
G = W // 2
LANES = 16
GROUPS = G // LANES
SCALE = 2.0 / 223.0
HALF = 112.0


def _floor_i32(x):
    t = x.astype(jnp.int32)
    tf = t.astype(jnp.float32)
    return jnp.where(tf > x, t - 1, t)


def _bf16r(x):
    u = lax.bitcast_convert_type(x, jnp.int32)
    r = (u + 0x7FFF + ((u >> 16) & 1)) & jnp.int32(-65536)
    return lax.bitcast_convert_type(r, jnp.float32)


_MESH = plsc.VectorSubcoreMesh(core_axis_name="c", subcore_axis_name="s")


@functools.partial(
    pl.kernel,
    mesh=_MESH,
    out_type=jax.ShapeDtypeStruct((N, C), jnp.float32),
    compiler_params=pltpu.CompilerParams(use_tc_tiling_on_sc=False),
    scratch_types=[
        pltpu.VMEM((80,), jnp.float32),
        pltpu.VMEM((2, G), jnp.int32),
        pltpu.VMEM((2, G), jnp.int32),
        pltpu.VMEM((2, G), jnp.int32),
        pltpu.VMEM((2, G), jnp.int32),
        pltpu.VMEM((W,), jnp.float32),
        pltpu.VMEM((W,), jnp.float32),
        pltpu.VMEM((W,), jnp.float32),
        pltpu.VMEM((W,), jnp.float32),
        pltpu.VMEM((G, C), jnp.float32),
        pltpu.VMEM((G, C), jnp.float32),
        pltpu.VMEM((G, C), jnp.float32),
        pltpu.VMEM((G, C), jnp.float32),
        pltpu.VMEM((G, C), jnp.float32),
        pltpu.SemaphoreType.DMA,
    ],
)
def _stn(img_hbm, theta_hbm, out_hbm,
         theta_v, ia_v, ib_v, ic_v, id_v,
         wa_v, wb_v, wc_v, wd_v,
         ra_v, rb_v, rc_v, rd_v, out_v, sem):
    wid = lax.axis_index("c") * 16 + lax.axis_index("s")
    b = wid // 4
    j0 = (wid % 4) * ROWS_PER_TILE
    bbase = b * HW

    pltpu.sync_copy(theta_hbm, theta_v.at[pl.ds(0, 64)])

    tvec = theta_v[pl.ds(b * 8, LANES)]
    t0, t1, t2, t3, t4, t5 = (
        _bf16r(jnp.full((LANES,), tvec[q], jnp.float32)) for q in range(6))

    iota_f = lax.iota(jnp.int32, LANES).astype(jnp.float32)

    def row_body(rr, carry):
        j = j0 + rr
        ynb = _bf16r(
            jnp.full((LANES,), j, jnp.int32).astype(jnp.float32) * SCALE - 1.0)
        cx = t1 * ynb + t2
        cy = t4 * ynb + t5

        for h in range(2):
            for g in range(GROUPS):
                base_i = h * G + g * LANES
                xnb = _bf16r((iota_f + float(base_i)) * SCALE - 1.0)
                xs = (t0 * xnb + cx + 1.0) * HALF
                ys = (t3 * xnb + cy + 1.0) * HALF
                x0 = _floor_i32(xs)
                y0 = _floor_i32(ys)
                x1 = x0 + 1
                y1 = y0 + 1
                x0c = jnp.clip(x0, 0, W - 1)
                x1c = jnp.clip(x1, 0, W - 1)
                y0c = jnp.clip(y0, 0, H - 1)
                y1c = jnp.clip(y1, 0, H - 1)
                x0f = x0c.astype(jnp.float32)
                x1f = x1c.astype(jnp.float32)
                y0f = y0c.astype(jnp.float32)
                y1f = y1c.astype(jnp.float32)
                dx1 = x1f - xs
                dx0 = xs - x0f
                dy1 = y1f - ys
                dy0 = ys - y0f
                rowa = bbase + y0c * W
                rowb = bbase + y1c * W
                sl = pl.ds(g * LANES, LANES)
                ia_v[h, sl] = rowa + x0c
                ib_v[h, sl] = rowb + x0c
                ic_v[h, sl] = rowa + x1c
                id_v[h, sl] = rowb + x1c
                wsl = pl.ds(base_i, LANES)
                wa_v[wsl] = dx1 * dy1
                wb_v[wsl] = dx1 * dy0
                wc_v[wsl] = dx0 * dy1
                wd_v[wsl] = dx0 * dy0

            cpa = pltpu.async_copy(img_hbm.at[ia_v.at[h]], ra_v, sem)
            cpb = pltpu.async_copy(img_hbm.at[ib_v.at[h]], rb_v, sem)
            cpc = pltpu.async_copy(img_hbm.at[ic_v.at[h]], rc_v, sem)
            cpd = pltpu.async_copy(img_hbm.at[id_v.at[h]], rd_v, sem)
            cpa.wait()
            cpb.wait()
            cpc.wait()
            cpd.wait()

            def blend_group(gg, bc):
                base = h * G + gg * LANES
                wa_grp = wa_v[pl.ds(base, LANES)]
                wb_grp = wb_v[pl.ds(base, LANES)]
                wc_grp = wc_v[pl.ds(base, LANES)]
                wd_grp = wd_v[pl.ds(base, LANES)]
                for e in range(LANES):
                    k = gg * LANES + e
                    wa = jnp.full((LANES,), wa_grp[e], jnp.float32)
                    wb = jnp.full((LANES,), wb_grp[e], jnp.float32)
                    wc = jnp.full((LANES,), wc_grp[e], jnp.float32)
                    wd = jnp.full((LANES,), wd_grp[e], jnp.float32)
                    for s in range(C // LANES):
                        csl = pl.ds(s * LANES, LANES)
                        out_v[k, csl] = (wa * ra_v[k, csl] + wb * rb_v[k, csl]
                                         + wc * rc_v[k, csl] + wd * rd_v[k, csl])
                return bc

            lax.fori_loop(0, GROUPS, blend_group, 0)
            p0 = bbase + j * W + h * G
            pltpu.sync_copy(out_v, out_hbm.at[pl.ds(p0, G)])
        return carry

    lax.fori_loop(0, ROWS_PER_TILE, row_body, 0)


def kernel(images, theta):
    img_flat = images.reshape(N, C)
    theta_pad = jnp.pad(theta, ((0, 0), (0, 2))).reshape(64)
    out = _stn(img_flat, theta_pad)
    return out.reshape(B, H, W, C)
